# baseline (speedup 1.0000x reference)
import numpy as np

import jax
import jax.numpy as jnp
from jax.experimental import pallas as pl
from jax.experimental.pallas import tpu as pltpu

_B, _C, _F = 2, 64, 64
_NLAT_IN, _NLON_IN = 64, 128
_NLAT_OUT, _NLON_OUT = 32, 64
_K = 3
_ROWS = 6
_NCHUNK = 8
_CH = _NLAT_IN * _NLON_IN // _NCHUNK

_NT = (((1,), (1,)), ((), ()))


def _row_start(t: int) -> int:
    return min(max(2 * t - 2, 0), _NLAT_IN - _ROWS)


def _disco_kernel(psw_ref, wbd_ref, b_ref, x_hbm, out_ref, xs_ref, sems):
    lc = _NLAT_IN * _NLON_IN // _NCHUNK
    copies = [
        pltpu.make_async_copy(
            x_hbm.at[c * lc:(c + 1) * lc, :],
            xs_ref.at[c * lc:(c + 1) * lc, :],
            sems.at[c])
        for c in range(_NCHUNK)
    ]
    for cp in copies:
        cp.start()
    waited = 0
    for t in range(_NLAT_OUT):
        i0 = _row_start(t)
        need = (i0 + _ROWS + 7) // 8
        while waited < need:
            copies[waited].wait()
            waited += 1
        kblocks = []
        for k in range(_K):
            rs = []
            for r in range(_ROWS):
                v = psw_ref[k, t, r, :]
                ct0 = jnp.broadcast_to(v[None, :], (_NLON_OUT, _NLON_IN))
                rs.append(pltpu.roll(ct0, 0, axis=1, stride=2, stride_axis=0))
            kblocks.append(jnp.concatenate(rs, axis=1))
        ct = jnp.concatenate(kblocks, axis=0)
        xw = xs_ref[i0 * _NLON_IN:i0 * _NLON_IN + _ROWS * _NLON_IN, :]
        y = jax.lax.dot(ct, xw,
                        preferred_element_type=jnp.float32)
        ob = None
        for k in range(_K):
            q = jax.lax.dot_general(wbd_ref[k],
                                    y[k * _NLON_OUT:(k + 1) * _NLON_OUT, :],
                                    _NT,
                                    preferred_element_type=jnp.float32)
            ob = q if ob is None else ob + q
        out_ref[:, t, :] = ob + b_ref[:, :]


def kernel(x, psi, quad_weights, weight, bias):
    xf = x.reshape(_B * _C, _NLAT_IN * _NLON_IN).T
    psiR = psi.reshape(_K, _NLAT_OUT, _NLAT_IN, _NLON_IN)
    starts = np.array([_row_start(t) for t in range(_NLAT_OUT)])
    idx = jnp.asarray(starts[:, None] + np.arange(_ROWS)[None, :])
    psw = jnp.take_along_axis(psiR, idx[None, :, :, None], axis=2)
    psw = psw * quad_weights[idx, 0][None, :, :, None]
    eyeb = jnp.eye(_B, dtype=jnp.float32)
    wbdT = jnp.einsum('fck,ab->kafbc', weight, eyeb).reshape(
        _K, _B * _F, _B * _C)
    br = jnp.tile(bias, _B).reshape(_B * _F, 1)
    out = pl.pallas_call(
        _disco_kernel,
        grid=(1,),
        in_specs=[
            pl.BlockSpec((_K, _NLAT_OUT, _ROWS, _NLON_IN),
                         lambda s: (0, 0, 0, 0)),
            pl.BlockSpec((_K, _B * _F, _B * _C), lambda s: (0, 0, 0)),
            pl.BlockSpec((_B * _F, 1), lambda s: (0, 0)),
            pl.BlockSpec(memory_space=pltpu.MemorySpace.HBM),
        ],
        out_specs=pl.BlockSpec((_B * _F, _NLAT_OUT, _NLON_OUT),
                               lambda s: (0, 0, 0)),
        out_shape=jax.ShapeDtypeStruct((_B * _F, _NLAT_OUT, _NLON_OUT),
                                       jnp.float32),
        scratch_shapes=[
            pltpu.VMEM((_NLAT_IN * _NLON_IN, _B * _C), jnp.float32),
            pltpu.SemaphoreType.DMA((_NCHUNK,)),
        ],
    )(psw, wbdT, br, xf)
    return out.reshape(_B, _F, _NLAT_OUT, _NLON_OUT)


# speedup vs baseline: 1.1199x; 1.1199x over previous
import numpy as np

import jax
import jax.numpy as jnp
from jax.experimental import pallas as pl
from jax.experimental.pallas import tpu as pltpu

_B, _C, _F = 2, 64, 64
_NLAT_IN, _NLON_IN = 64, 128
_NLAT_OUT, _NLON_OUT = 32, 64
_K = 3
_ROWS = 6

_NT = (((1,), (1,)), ((), ()))


def _row_start(t: int) -> int:
    return min(max(2 * t - 2, 0), _NLAT_IN - _ROWS)


def _disco_kernel(psw_ref, wbd_ref, b_ref, x_ref, out_ref):
    for t in range(_NLAT_OUT):
        i0 = _row_start(t)
        kblocks = []
        for k in range(_K):
            rs = []
            for r in range(_ROWS):
                v = psw_ref[k, t, r, :]
                ct0 = jnp.broadcast_to(v[None, :], (_NLON_OUT, _NLON_IN))
                rs.append(pltpu.roll(ct0, 0, axis=1, stride=2, stride_axis=0))
            kblocks.append(jnp.concatenate(rs, axis=1))
        ct = jnp.concatenate(kblocks, axis=0)
        xw = x_ref[:, i0 * _NLON_IN:i0 * _NLON_IN + _ROWS * _NLON_IN]
        y = jax.lax.dot_general(ct, xw, _NT,
                                preferred_element_type=jnp.float32)
        ob = None
        for k in range(_K):
            q = jax.lax.dot_general(wbd_ref[k],
                                    y[k * _NLON_OUT:(k + 1) * _NLON_OUT, :],
                                    _NT,
                                    preferred_element_type=jnp.float32)
            ob = q if ob is None else ob + q
        out_ref[:, t, :] = ob + b_ref[:, :]


def kernel(x, psi, quad_weights, weight, bias):
    xf = x.reshape(_B * _C, _NLAT_IN * _NLON_IN)
    psiR = psi.reshape(_K, _NLAT_OUT, _NLAT_IN, _NLON_IN)
    starts = np.array([_row_start(t) for t in range(_NLAT_OUT)])
    idx = jnp.asarray(starts[:, None] + np.arange(_ROWS)[None, :])
    psw = jnp.take_along_axis(psiR, idx[None, :, :, None], axis=2)
    psw = psw * quad_weights[idx, 0][None, :, :, None]
    eyeb = jnp.eye(_B, dtype=jnp.float32)
    wbdT = jnp.einsum('fck,ab->kafbc', weight, eyeb).reshape(
        _K, _B * _F, _B * _C)
    br = jnp.tile(bias, _B).reshape(_B * _F, 1)
    out = pl.pallas_call(
        _disco_kernel,
        grid=(1,),
        in_specs=[
            pl.BlockSpec((_K, _NLAT_OUT, _ROWS, _NLON_IN),
                         lambda s: (0, 0, 0, 0)),
            pl.BlockSpec((_K, _B * _F, _B * _C), lambda s: (0, 0, 0)),
            pl.BlockSpec((_B * _F, 1), lambda s: (0, 0)),
            pl.BlockSpec((_B * _C, _NLAT_IN * _NLON_IN), lambda s: (0, 0)),
        ],
        out_specs=pl.BlockSpec((_B * _F, _NLAT_OUT, _NLON_OUT),
                               lambda s: (0, 0, 0)),
        out_shape=jax.ShapeDtypeStruct((_B * _F, _NLAT_OUT, _NLON_OUT),
                                       jnp.float32),
    )(psw, wbdT, br, xf)
    return out.reshape(_B, _F, _NLAT_OUT, _NLON_OUT)


# speedup vs baseline: 1.2090x; 1.0796x over previous
import numpy as np

import jax
import jax.numpy as jnp
from jax.experimental import pallas as pl
from jax.experimental.pallas import tpu as pltpu

_B, _C, _F = 2, 64, 64
_NLAT_IN, _NLON_IN = 64, 128
_NLAT_OUT, _NLON_OUT = 32, 64
_K = 3
_ROWS = 6

_NT = (((1,), (1,)), ((), ()))


def _row_start(t: int) -> int:
    return min(max(2 * t - 2, 0), _NLAT_IN - _ROWS)


def _disco_kernel(psw_ref, wbd_ref, b_ref, x_ref, out_ref):
    for t in range(_NLAT_OUT):
        i0 = _row_start(t)
        kblocks = []
        for k in range(_K):
            rs = []
            for r in range(_ROWS):
                v = psw_ref[k, t, r, :]
                ct0 = jnp.broadcast_to(v[None, :], (_NLON_OUT, _NLON_IN))
                rs.append(pltpu.roll(ct0, 0, axis=1, stride=2, stride_axis=0))
            kblocks.append(jnp.concatenate(rs, axis=1))
        ct = jnp.concatenate(kblocks, axis=0)
        ctb = ct.astype(jnp.bfloat16)
        xw = x_ref[:, i0 * _NLON_IN:i0 * _NLON_IN + _ROWS * _NLON_IN]
        y = jax.lax.dot_general(ctb, xw, _NT,
                                preferred_element_type=jnp.float32)
        ob = None
        for k in range(_K):
            q = jax.lax.dot_general(wbd_ref[k],
                                    y[k * _NLON_OUT:(k + 1) * _NLON_OUT, :],
                                    _NT,
                                    preferred_element_type=jnp.float32)
            ob = q if ob is None else ob + q
        out_ref[:, t, :] = ob + b_ref[:, :]


def kernel(x, psi, quad_weights, weight, bias):
    xf = x.reshape(_B * _C, _NLAT_IN * _NLON_IN).astype(jnp.bfloat16)
    psiR = psi.reshape(_K, _NLAT_OUT, _NLAT_IN, _NLON_IN)
    starts = np.array([_row_start(t) for t in range(_NLAT_OUT)])
    idx = jnp.asarray(starts[:, None] + np.arange(_ROWS)[None, :])
    psw = jnp.take_along_axis(psiR, idx[None, :, :, None], axis=2)
    psw = psw * quad_weights[idx, 0][None, :, :, None]
    eyeb = jnp.eye(_B, dtype=jnp.float32)
    wbdT = jnp.einsum('fck,ab->kafbc', weight, eyeb).reshape(
        _K, _B * _F, _B * _C)
    br = jnp.tile(bias, _B).reshape(_B * _F, 1)
    out = pl.pallas_call(
        _disco_kernel,
        grid=(1,),
        in_specs=[
            pl.BlockSpec((_K, _NLAT_OUT, _ROWS, _NLON_IN),
                         lambda s: (0, 0, 0, 0)),
            pl.BlockSpec((_K, _B * _F, _B * _C), lambda s: (0, 0, 0)),
            pl.BlockSpec((_B * _F, 1), lambda s: (0, 0)),
            pl.BlockSpec((_B * _C, _NLAT_IN * _NLON_IN), lambda s: (0, 0)),
        ],
        out_specs=pl.BlockSpec((_B * _F, _NLAT_OUT, _NLON_OUT),
                               lambda s: (0, 0, 0)),
        out_shape=jax.ShapeDtypeStruct((_B * _F, _NLAT_OUT, _NLON_OUT),
                                       jnp.float32),
    )(psw, wbdT, br, xf)
    return out.reshape(_B, _F, _NLAT_OUT, _NLON_OUT)
